# baseline (speedup 1.0000x reference)
import functools

import jax
import jax.numpy as jnp
from jax import lax
from jax.experimental import pallas as pl
from jax.experimental.pallas import tpu as pltpu
from jax.experimental.pallas import tpu_sc as plsc

EMBED_DIM = 64
OUT_DIM = 128

_NC, _NS = 2, 16
_NW = _NC * _NS
_CHUNK = 512


def _sc_gather(table, idx, n_rows):
    rows_per_w = n_rows // _NW
    n_chunks = rows_per_w // _CHUNK
    mesh = plsc.VectorSubcoreMesh(core_axis_name="c", subcore_axis_name="s")

    @functools.partial(
        pl.kernel,
        mesh=mesh,
        out_type=jax.ShapeDtypeStruct((n_rows, EMBED_DIM), jnp.float32),
        scratch_types=[
            pltpu.VMEM((_CHUNK,), jnp.int32),
            pltpu.VMEM((_CHUNK, EMBED_DIM), jnp.float32),
            pltpu.SemaphoreType.DMA,
        ],
        compiler_params=pltpu.CompilerParams(use_tc_tiling_on_sc=False),
    )
    def gather_kernel(table_hbm, idx_hbm, out_hbm, idx_v, rows_v, sem):
        wid = lax.axis_index("s") * _NC + lax.axis_index("c")
        base = wid * rows_per_w

        def body(i, carry):
            off = base + i * _CHUNK
            pltpu.sync_copy(idx_hbm.at[pl.ds(off, _CHUNK)], idx_v)
            pltpu.async_copy(table_hbm.at[idx_v], rows_v, sem).wait()
            pltpu.sync_copy(rows_v, out_hbm.at[pl.ds(off, _CHUNK)])
            return carry

        lax.fori_loop(0, n_chunks, body, 0)

    return gather_kernel(table, idx)


_SQRT_HALF = 0.7071067811865476


def _proj_body(e_ref, w_ref, b_ref, o_ref):
    x = e_ref[...]
    h = 0.5 * x * (1.0 + lax.erf(x * _SQRT_HALF))
    acc = lax.dot_general(
        h, w_ref[...], (((1,), (1,)), ((), ())),
        preferred_element_type=jnp.float32,
    )
    o_ref[...] = acc + b_ref[...]


def _tc_project(e, w, b2d, n_rows, blk):
    grid = (n_rows // blk,)
    return pl.pallas_call(
        _proj_body,
        grid=grid,
        in_specs=[
            pl.BlockSpec((blk, EMBED_DIM), lambda i: (i, 0)),
            pl.BlockSpec((OUT_DIM, EMBED_DIM), lambda i: (0, 0)),
            pl.BlockSpec((1, OUT_DIM), lambda i: (0, 0)),
        ],
        out_specs=pl.BlockSpec((blk, OUT_DIM), lambda i: (i, 0)),
        out_shape=jax.ShapeDtypeStruct((n_rows, OUT_DIM), jnp.float32),
    )(e, w, b2d)


def kernel(times, table, W, b):
    bsz, seq = times.shape
    n_rows = bsz * seq
    idx = times.reshape(n_rows).astype(jnp.int32)
    e = _sc_gather(table, idx, n_rows)
    out = _tc_project(e, W, b.reshape(1, OUT_DIM), n_rows, 2048)
    return out.reshape(bsz, seq, OUT_DIM)


# speedup vs baseline: 1.4224x; 1.4224x over previous
import functools

import jax
import jax.numpy as jnp
from jax import lax
from jax.experimental import pallas as pl
from jax.experimental.pallas import tpu as pltpu
from jax.experimental.pallas import tpu_sc as plsc

EMBED_DIM = 64
OUT_DIM = 128

_NC, _NS = 2, 16
_NW = _NC * _NS
_CHUNK = 512


_CHUNKR = 4


def _sc_gather(table, times2d):
    bsz, seq = times2d.shape
    tr_per_w = bsz // _NW
    n_chunks = tr_per_w // _CHUNKR
    mesh = plsc.VectorSubcoreMesh(core_axis_name="c", subcore_axis_name="s")

    @functools.partial(
        pl.kernel,
        mesh=mesh,
        out_type=jax.ShapeDtypeStruct((bsz, seq, 2 * EMBED_DIM), jnp.float32),
        scratch_types=[
            pltpu.VMEM((_CHUNKR, seq), jnp.int32),
            pltpu.VMEM((_CHUNKR, seq, EMBED_DIM), jnp.float32),
            pltpu.SemaphoreType.DMA,
        ],
        compiler_params=pltpu.CompilerParams(use_tc_tiling_on_sc=False),
    )
    def gather_kernel(table_hbm, times_hbm, out_hbm, idx_v, rows_v, sem):
        wid = lax.axis_index("s") * _NC + lax.axis_index("c")
        base = wid * tr_per_w

        def body(i, carry):
            r0 = base + i * _CHUNKR
            pltpu.sync_copy(times_hbm.at[pl.ds(r0, _CHUNKR)], idx_v)
            copies = [
                pltpu.async_copy(
                    table_hbm.at[idx_v.at[k]], rows_v.at[k], sem
                )
                for k in range(_CHUNKR)
            ]
            for c in copies:
                c.wait()
            pltpu.sync_copy(
                rows_v,
                out_hbm.at[pl.ds(r0, _CHUNKR), :, pl.ds(0, EMBED_DIM)],
            )
            return carry

        lax.fori_loop(0, n_chunks, body, 0)

    return gather_kernel(table, times2d)


_SQRT_HALF = 0.7071067811865476


def _proj_body(e_ref, w_ref, b_ref, o_ref):
    x = e_ref[...][:, :EMBED_DIM]
    h = 0.5 * x * (1.0 + lax.erf(x * _SQRT_HALF))
    acc = lax.dot_general(
        h, w_ref[...], (((1,), (1,)), ((), ())),
        preferred_element_type=jnp.float32,
    )
    o_ref[...] = acc + b_ref[...]


def _tc_project(e, w, b2d, n_rows, blk):
    grid = (n_rows // blk,)
    return pl.pallas_call(
        _proj_body,
        grid=grid,
        in_specs=[
            pl.BlockSpec((blk, 2 * EMBED_DIM), lambda i: (i, 0)),
            pl.BlockSpec((OUT_DIM, EMBED_DIM), lambda i: (0, 0)),
            pl.BlockSpec((1, OUT_DIM), lambda i: (0, 0)),
        ],
        out_specs=pl.BlockSpec((blk, OUT_DIM), lambda i: (i, 0)),
        out_shape=jax.ShapeDtypeStruct((n_rows, OUT_DIM), jnp.float32),
    )(e, w, b2d)


def kernel(times, table, W, b):
    bsz, seq = times.shape
    n_rows = bsz * seq
    e3 = _sc_gather(table, times.astype(jnp.int32))
    e = e3.reshape(n_rows, 2 * EMBED_DIM)
    out = _tc_project(e, W, b.reshape(1, OUT_DIM), n_rows, 4096)
    return out.reshape(bsz, seq, OUT_DIM)
